# Initial kernel scaffold; baseline (speedup 1.0000x reference)
import jax
import jax.numpy as jnp
from jax.experimental import pallas as pl


def kernel(x, edge_index, W1, att_src1, att_dst1, b1, W2, att_src2, att_dst2, b2):
    raise NotImplementedError("write your pallas kernel here")


# speedup vs baseline: 20.7117x; 20.7117x over previous
import functools

import jax
import jax.numpy as jnp
from jax import lax
from jax.experimental import pallas as pl
from jax.experimental.pallas import tpu as pltpu
from jax.experimental.pallas import tpu_sc as plsc

NC = 2
NS = 16
LANES = 16
WIN = 128


def _round_up(v, m):
    return (v + m - 1) // m * m


_GDN = lax.GatherDimensionNumbers(
    offset_dims=(), collapsed_slice_dims=(0,), start_index_map=(0,))


def _lane_perm(v, perm):
    return lax.gather(v, perm[:, None], _GDN, (1,),
                      mode=lax.GatherScatterMode.PROMISE_IN_BOUNDS)


def _lane_max(v):
    idx = lax.iota(jnp.int32, LANES)
    for sh in (8, 4, 2, 1):
        v = jnp.maximum(v, _lane_perm(v, jnp.bitwise_and(idx + sh, LANES - 1)))
    return v


def _segment_add_16(den_ref, keys, vals):
    lid = lax.iota(jnp.int32, LANES)
    sk, sv = plsc.sort_key_val(keys, vals)
    cum = plsc.cumsum(sv)
    pk = _lane_perm(sk, jnp.maximum(lid - 1, 0))
    isfirst = jnp.logical_or(lid == 0, sk != pk)
    nk = _lane_perm(sk, jnp.minimum(lid + 1, LANES - 1))
    islast = jnp.logical_or(lid == LANES - 1, sk != nk)
    runfirst = plsc.cummax(jnp.where(isfirst, lid, 0))
    prevcum = _lane_perm(cum, jnp.maximum(runfirst - 1, 0))
    prevcum = jnp.where(runfirst == 0, jnp.zeros_like(prevcum), prevcum)
    plsc.addupdate_scatter(den_ref, [sk], cum - prevcum, mask=islast)


def _tc_embed(xp, w, att_s, att_d):
    npad = xp.shape[0]
    cdim = w.shape[1]
    hc = cdim // NC

    def body(x_ref, w_ref, s_ref, d_ref, h_ref, as_ref, ad_ref):
        h = jnp.dot(x_ref[...], w_ref[...], preferred_element_type=jnp.float32)
        h_ref[0] = h[:, :hc]
        h_ref[1] = h[:, hc:]
        as_ref[...] = jnp.sum(h * s_ref[...][None, :], axis=1)
        ad_ref[...] = jnp.sum(h * d_ref[...][None, :], axis=1)

    return pl.pallas_call(
        body,
        out_shape=[
            jax.ShapeDtypeStruct((NC, npad, hc), jnp.float32),
            jax.ShapeDtypeStruct((npad,), jnp.float32),
            jax.ShapeDtypeStruct((npad,), jnp.float32),
        ],
    )(xp, w, att_s, att_d)


def _tc_mid(parts, b, w, att_s, att_d):
    npad = parts.shape[1]
    cdim = w.shape[1]
    hc = cdim // NC

    def body(p_ref, b_ref, w_ref, s_ref, d_ref, h_ref, as_ref, ad_ref):
        g = jnp.concatenate([p_ref[0], p_ref[1]], axis=1)
        x = jax.nn.relu(g + b_ref[...][None, :])
        h = jnp.dot(x, w_ref[...], preferred_element_type=jnp.float32)
        h_ref[0] = h[:, :hc]
        h_ref[1] = h[:, hc:]
        as_ref[...] = jnp.sum(h * s_ref[...][None, :], axis=1)
        ad_ref[...] = jnp.sum(h * d_ref[...][None, :], axis=1)

    return pl.pallas_call(
        body,
        out_shape=[
            jax.ShapeDtypeStruct((NC, npad, hc), jnp.float32),
            jax.ShapeDtypeStruct((npad,), jnp.float32),
            jax.ShapeDtypeStruct((npad,), jnp.float32),
        ],
    )(parts, b, w, att_s, att_d)


def _tc_out(parts, b):
    _, npad, hcdim = parts.shape
    cdim = NC * hcdim

    def body(p_ref, b_ref, o_ref):
        o = jnp.concatenate([p_ref[0], p_ref[1]], axis=1) + b_ref[...][None, :]
        m = jnp.max(o, axis=1, keepdims=True)
        z = o - m
        o_ref[...] = z - jnp.log(jnp.sum(jnp.exp(z), axis=1, keepdims=True))

    return pl.pallas_call(
        body,
        out_shape=jax.ShapeDtypeStruct((npad, cdim), jnp.float32),
    )(parts, b)


def _sc_denom(asv, adv, src2, dst2):
    npad = asv.shape[0]
    nw, wpt, _ = src2.shape
    npt = npad // NS

    mesh = plsc.VectorSubcoreMesh(core_axis_name="c", subcore_axis_name="s")

    @functools.partial(
        pl.kernel,
        out_type=[
            jax.ShapeDtypeStruct((nw, wpt, WIN), jnp.float32),
            jax.ShapeDtypeStruct((nw, npad), jnp.float32),
        ],
        mesh=mesh,
        compiler_params=pltpu.CompilerParams(needs_layout_passes=False, use_tc_tiling_on_sc=False),
        scratch_types=[
            pltpu.VMEM((npad,), jnp.float32),
            pltpu.VMEM((npad,), jnp.float32),
            pltpu.VMEM((wpt, WIN), jnp.int32),
            pltpu.VMEM((wpt, WIN), jnp.int32),
            pltpu.VMEM((wpt, WIN), jnp.float32),
            pltpu.VMEM((npad,), jnp.float32),
        ],
    )
    def k(asv_h, adv_h, src_h, dst_h, p_h, dp_h,
          asv_v, adv_v, src_v, dst_v, p_v, den_v):
        c = lax.axis_index("c")
        s = lax.axis_index("s")
        wid = s * NC + c
        pltpu.sync_copy(asv_h, asv_v)
        pltpu.sync_copy(adv_h, adv_v)
        pltpu.sync_copy(src_h.at[wid], src_v)
        pltpu.sync_copy(dst_h.at[wid], dst_v)

        def zero(i, carry):
            den_v[pl.ds(i * LANES, LANES)] = jnp.zeros((LANES,), jnp.float32)
            return carry
        lax.fori_loop(0, npad // LANES, zero, 0)

        def mx(i, acc):
            return jnp.maximum(acc, asv_v[pl.ds(i * LANES, LANES)])
        acc = lax.fori_loop(0, npad // LANES, mx,
                            jnp.full((LANES,), -jnp.inf, jnp.float32))
        a_top = _lane_max(acc)

        def win(j, carry):
            def grp(g, carry2):
                sl = pl.ds(g * LANES, LANES)
                si = src_v[j, sl]
                di = dst_v[j, sl]
                a_s = plsc.load_gather(asv_v, [si])
                a_d = plsc.load_gather(adv_v, [di])
                t = a_s + a_d
                e = jnp.maximum(t, 0.2 * t)
                u = a_top + a_d
                mp = jnp.maximum(u, 0.2 * u)
                p16 = jnp.exp(e - mp)
                p_v[j, sl] = p16
                _segment_add_16(den_v, di, p16)
                return carry2
            lax.fori_loop(0, WIN // LANES, grp, 0)
            return carry
        lax.fori_loop(0, wpt, win, 0)

        pltpu.sync_copy(p_v, p_h.at[wid])
        pltpu.sync_copy(den_v, dp_h.at[wid])

    return k(asv, adv, src2, dst2)


def _sc_aggregate(hsplit, dparts, p2, src2, dst2):
    _, npad, hc = hsplit.shape
    ns, wpt, _ = src2.shape
    cl = hc // LANES
    nfull = npad // WIN
    tail = npad - nfull * WIN
    rounds = _round_up(nfull + (1 if tail else 0), NS) // NS

    mesh = plsc.VectorSubcoreMesh(core_axis_name="c", subcore_axis_name="s")

    @functools.partial(
        pl.kernel,
        out_type=jax.ShapeDtypeStruct((NC, npad, hc), jnp.float32),
        mesh=mesh,
        compiler_params=pltpu.CompilerParams(needs_layout_passes=False, use_tc_tiling_on_sc=False),
        scratch_types=[
            pltpu.VMEM((NC * NS, WIN), jnp.float32),
            pltpu.VMEM((wpt, WIN), jnp.int32),
            pltpu.VMEM((wpt, WIN), jnp.int32),
            pltpu.VMEM((wpt, WIN), jnp.float32),
            pltpu.VMEM((WIN, hc), jnp.float32),
            pltpu.VMEM_SHARED((npad, hc), jnp.float32),
        ],
    )
    def k(h_h, dp_h, p_h, src_h, dst_h, out_h,
          dpw_v, src_v, dst_v, p_v, rows_v, acc_sp):
        c = lax.axis_index("c")
        s = lax.axis_index("s")

        pltpu.sync_copy(src_h.at[s], src_v)
        pltpu.sync_copy(dst_h.at[s], dst_v)
        pltpu.sync_copy(p_h.at[s], p_v)

        def zrow(r, carry):
            for q in range(cl):
                rows_v[r, pl.ds(q * LANES, LANES)] = jnp.zeros(
                    (LANES,), jnp.float32)
            return carry
        lax.fori_loop(0, WIN, zrow, 0)

        for i in range(rounds):
            ch = s + NS * i

            @pl.when(ch < nfull)
            def _():
                pltpu.sync_copy(rows_v, acc_sp.at[pl.ds(ch * WIN, WIN)])
        if tail:
            @pl.when(s == NS - 1)
            def _():
                pltpu.sync_copy(rows_v.at[pl.ds(0, tail)],
                                acc_sp.at[pl.ds(nfull * WIN, tail)])
        plsc.subcore_barrier()

        def win(j, carry):
            pltpu.sync_copy(h_h.at[c].at[src_v.at[j]], rows_v)

            def rowscale(g, carry2):
                p16 = p_v[j, pl.ds(g * LANES, LANES)]
                for r in range(LANES):
                    a = p16[r]
                    row = g * LANES + r
                    for q in range(cl):
                        sl = pl.ds(q * LANES, LANES)
                        rows_v[row, sl] = rows_v[row, sl] * a
                return carry2
            lax.fori_loop(0, WIN // LANES, rowscale, 0)
            pltpu.sync_copy(rows_v, acc_sp.at[dst_v.at[j]], add=True)
            return carry
        lax.fori_loop(0, wpt, win, 0)
        plsc.subcore_barrier()

        def read_chunk(base, nrows):
            sl = pl.ds(base, nrows)
            pltpu.sync_copy(acc_sp.at[sl], rows_v.at[pl.ds(0, nrows)])
            pltpu.sync_copy(dp_h.at[:, sl], dpw_v.at[:, pl.ds(0, nrows)])

            def divgrp(g, carry2):
                gsl = pl.ds(g * LANES, LANES)

                def racc(r, acc):
                    return acc + dpw_v[r, gsl]
                d16 = lax.fori_loop(0, NC * NS, racc,
                                    jnp.zeros((LANES,), jnp.float32))
                inv = 1.0 / (d16 + 1e-16)
                for r in range(LANES):
                    iv = inv[r]
                    row = g * LANES + r
                    for q in range(cl):
                        qsl = pl.ds(q * LANES, LANES)
                        rows_v[row, qsl] = rows_v[row, qsl] * iv
                return carry2
            lax.fori_loop(0, nrows // LANES, divgrp, 0)
            pltpu.sync_copy(rows_v.at[pl.ds(0, nrows)], out_h.at[c, sl])

        for i in range(rounds):
            ch = s + NS * i

            @pl.when(ch < nfull)
            def _():
                read_chunk(ch * WIN, WIN)
        if tail:
            @pl.when(s == NS - 1)
            def _():
                read_chunk(nfull * WIN, tail)

    return k(hsplit, dparts, p2, src2, dst2)


def kernel(x, edge_index, W1, att_src1, att_dst1, b1,
           W2, att_src2, att_dst2, b2):
    n, _ = x.shape
    e = edge_index.shape[1]
    npad = _round_up(n + 1, 2 * LANES)
    epad = _round_up(e + n, NC * NS * WIN)
    nw = NC * NS
    wpt = epad // (nw * WIN)

    xp = jnp.zeros((npad, x.shape[1]), jnp.float32).at[:n].set(x)
    loop = jnp.arange(n, dtype=jnp.int32)
    pad = jnp.full((epad - e - n,), n, jnp.int32)
    src = jnp.concatenate(
        [edge_index[0].astype(jnp.int32), loop, pad]).reshape(nw, wpt, WIN)
    dst = jnp.concatenate(
        [edge_index[1].astype(jnp.int32), loop, pad]).reshape(nw, wpt, WIN)

    srcb = src.reshape(NS, nw * wpt // NS, WIN)
    dstb = dst.reshape(NS, nw * wpt // NS, WIN)

    h1, as1, ad1 = _tc_embed(xp, W1, att_src1[0], att_dst1[0])
    p1, dp1 = _sc_denom(as1, ad1, src, dst)
    acc1 = _sc_aggregate(h1, dp1, p1.reshape(NS, -1, WIN), srcb, dstb)

    h2, as2, ad2 = _tc_mid(acc1, b1, W2, att_src2[0], att_dst2[0])
    p2, dp2 = _sc_denom(as2, ad2, src, dst)
    acc2 = _sc_aggregate(h2, dp2, p2.reshape(NS, -1, WIN), srcb, dstb)

    out = _tc_out(acc2, b2)
    return out[:n]


# speedup vs baseline: 22.6430x; 1.0932x over previous
import functools

import jax
import jax.numpy as jnp
from jax import lax
from jax.experimental import pallas as pl
from jax.experimental.pallas import tpu as pltpu
from jax.experimental.pallas import tpu_sc as plsc

NC = 2
NS = 16
LANES = 16
WIN = 128


def _round_up(v, m):
    return (v + m - 1) // m * m


_GDN = lax.GatherDimensionNumbers(
    offset_dims=(), collapsed_slice_dims=(0,), start_index_map=(0,))


def _lane_perm(v, perm):
    return lax.gather(v, perm[:, None], _GDN, (1,),
                      mode=lax.GatherScatterMode.PROMISE_IN_BOUNDS)


def _lane_max(v):
    idx = lax.iota(jnp.int32, LANES)
    for sh in (8, 4, 2, 1):
        v = jnp.maximum(v, _lane_perm(v, jnp.bitwise_and(idx + sh, LANES - 1)))
    return v


def _segment_add_16(den_ref, keys, vals):
    lid = lax.iota(jnp.int32, LANES)
    sk, sv = plsc.sort_key_val(keys, vals)
    cum = plsc.cumsum(sv)
    pk = _lane_perm(sk, jnp.maximum(lid - 1, 0))
    isfirst = jnp.logical_or(lid == 0, sk != pk)
    nk = _lane_perm(sk, jnp.minimum(lid + 1, LANES - 1))
    islast = jnp.logical_or(lid == LANES - 1, sk != nk)
    runfirst = plsc.cummax(jnp.where(isfirst, lid, 0))
    prevcum = _lane_perm(cum, jnp.maximum(runfirst - 1, 0))
    prevcum = jnp.where(runfirst == 0, jnp.zeros_like(prevcum), prevcum)
    plsc.addupdate_scatter(den_ref, [sk], cum - prevcum, mask=islast)


def _tc_embed(xp, w, att_s, att_d):
    npad = xp.shape[0]
    cdim = w.shape[1]
    hc = cdim // NC

    def body(x_ref, w_ref, s_ref, d_ref, h_ref, as_ref, ad_ref):
        h = jnp.dot(x_ref[...], w_ref[...], preferred_element_type=jnp.float32)
        h_ref[0] = h[:, :hc]
        h_ref[1] = h[:, hc:]
        as_ref[...] = jnp.sum(h * s_ref[...][None, :], axis=1)
        ad_ref[...] = jnp.sum(h * d_ref[...][None, :], axis=1)

    return pl.pallas_call(
        body,
        out_shape=[
            jax.ShapeDtypeStruct((NC, npad, hc), jnp.float32),
            jax.ShapeDtypeStruct((npad,), jnp.float32),
            jax.ShapeDtypeStruct((npad,), jnp.float32),
        ],
    )(xp, w, att_s, att_d)


def _tc_mid(parts, b, w, att_s, att_d):
    npad = parts.shape[1]
    cdim = w.shape[1]
    hc = cdim // NC

    def body(p_ref, b_ref, w_ref, s_ref, d_ref, h_ref, as_ref, ad_ref):
        g = jnp.concatenate([p_ref[0], p_ref[1]], axis=1)
        x = jax.nn.relu(g + b_ref[...][None, :])
        h = jnp.dot(x, w_ref[...], preferred_element_type=jnp.float32)
        h_ref[0] = h[:, :hc]
        h_ref[1] = h[:, hc:]
        as_ref[...] = jnp.sum(h * s_ref[...][None, :], axis=1)
        ad_ref[...] = jnp.sum(h * d_ref[...][None, :], axis=1)

    return pl.pallas_call(
        body,
        out_shape=[
            jax.ShapeDtypeStruct((NC, npad, hc), jnp.float32),
            jax.ShapeDtypeStruct((npad,), jnp.float32),
            jax.ShapeDtypeStruct((npad,), jnp.float32),
        ],
    )(parts, b, w, att_s, att_d)


def _tc_out(parts, b):
    _, npad, hcdim = parts.shape
    cdim = NC * hcdim

    def body(p_ref, b_ref, o_ref):
        o = jnp.concatenate([p_ref[0], p_ref[1]], axis=1) + b_ref[...][None, :]
        m = jnp.max(o, axis=1, keepdims=True)
        z = o - m
        o_ref[...] = z - jnp.log(jnp.sum(jnp.exp(z), axis=1, keepdims=True))

    return pl.pallas_call(
        body,
        out_shape=jax.ShapeDtypeStruct((npad, cdim), jnp.float32),
    )(parts, b)


def _sc_denom(asv, adv, src2, dst2):
    npad = asv.shape[0]
    nw, wpt, _ = src2.shape
    npt = npad // NS

    mesh = plsc.VectorSubcoreMesh(core_axis_name="c", subcore_axis_name="s")

    @functools.partial(
        pl.kernel,
        out_type=[
            jax.ShapeDtypeStruct((nw, wpt, WIN), jnp.float32),
            jax.ShapeDtypeStruct((nw, npad), jnp.float32),
        ],
        mesh=mesh,
        compiler_params=pltpu.CompilerParams(needs_layout_passes=False, use_tc_tiling_on_sc=False),
        scratch_types=[
            pltpu.VMEM((npad,), jnp.float32),
            pltpu.VMEM((npad,), jnp.float32),
            pltpu.VMEM((wpt, WIN), jnp.int32),
            pltpu.VMEM((wpt, WIN), jnp.int32),
            pltpu.VMEM((wpt, WIN), jnp.float32),
            pltpu.VMEM((npad,), jnp.float32),
        ],
    )
    def k(asv_h, adv_h, src_h, dst_h, p_h, dp_h,
          asv_v, adv_v, src_v, dst_v, p_v, den_v):
        c = lax.axis_index("c")
        s = lax.axis_index("s")
        wid = s * NC + c
        pltpu.sync_copy(asv_h, asv_v)
        pltpu.sync_copy(adv_h, adv_v)
        pltpu.sync_copy(src_h.at[wid], src_v)
        pltpu.sync_copy(dst_h.at[wid], dst_v)

        def zero(i, carry):
            den_v[pl.ds(i * LANES, LANES)] = jnp.zeros((LANES,), jnp.float32)
            return carry
        lax.fori_loop(0, npad // LANES, zero, 0)

        def mx(i, acc):
            return jnp.maximum(acc, asv_v[pl.ds(i * LANES, LANES)])
        acc = lax.fori_loop(0, npad // LANES, mx,
                            jnp.full((LANES,), -jnp.inf, jnp.float32))
        a_top = _lane_max(acc)

        def win(j, carry):
            def grp(g, carry2):
                sl = pl.ds(g * LANES, LANES)
                si = src_v[j, sl]
                di = dst_v[j, sl]
                a_s = plsc.load_gather(asv_v, [si])
                a_d = plsc.load_gather(adv_v, [di])
                t = a_s + a_d
                e = jnp.maximum(t, 0.2 * t)
                u = a_top + a_d
                mp = jnp.maximum(u, 0.2 * u)
                p16 = jnp.exp(e - mp)
                p_v[j, sl] = p16
                _segment_add_16(den_v, di, p16)
                return carry2
            lax.fori_loop(0, WIN // LANES, grp, 0)
            return carry
        lax.fori_loop(0, wpt, win, 0)

        pltpu.sync_copy(p_v, p_h.at[wid])
        pltpu.sync_copy(den_v, dp_h.at[wid])

    return k(asv, adv, src2, dst2)


def _sc_aggregate(hsplit, dparts, p2, src2, dst2):
    _, npad, hc = hsplit.shape
    ns, wpt, _ = src2.shape
    cl = hc // LANES
    nfull = npad // WIN
    tail = npad - nfull * WIN
    rounds = _round_up(nfull + (1 if tail else 0), NS) // NS

    mesh = plsc.VectorSubcoreMesh(core_axis_name="c", subcore_axis_name="s")

    @functools.partial(
        pl.kernel,
        out_type=jax.ShapeDtypeStruct((NC, npad, hc), jnp.float32),
        mesh=mesh,
        compiler_params=pltpu.CompilerParams(needs_layout_passes=False, use_tc_tiling_on_sc=False),
        scratch_types=[
            pltpu.VMEM((NC * NS, WIN), jnp.float32),
            pltpu.VMEM((wpt, WIN), jnp.int32),
            pltpu.VMEM((wpt, WIN), jnp.int32),
            pltpu.VMEM((wpt, WIN), jnp.float32),
            pltpu.VMEM((WIN, hc), jnp.float32),
            pltpu.VMEM((WIN, hc), jnp.float32),
            pltpu.SemaphoreType.DMA,
            pltpu.SemaphoreType.DMA,
            pltpu.VMEM_SHARED((npad, hc), jnp.float32),
        ],
    )
    def k(h_h, dp_h, p_h, src_h, dst_h, out_h,
          dpw_v, src_v, dst_v, p_v, rows_v, rows2_v, ssem0, ssem1, acc_sp):
        c = lax.axis_index("c")
        s = lax.axis_index("s")

        pltpu.sync_copy(src_h.at[s], src_v)
        pltpu.sync_copy(dst_h.at[s], dst_v)
        pltpu.sync_copy(p_h.at[s], p_v)

        def zrow(r, carry):
            for q in range(cl):
                rows_v[r, pl.ds(q * LANES, LANES)] = jnp.zeros(
                    (LANES,), jnp.float32)
            return carry
        lax.fori_loop(0, WIN, zrow, 0)

        for i in range(rounds):
            ch = s + NS * i

            @pl.when(ch < nfull)
            def _():
                pltpu.sync_copy(rows_v, acc_sp.at[pl.ds(ch * WIN, WIN)])
        if tail:
            @pl.when(s == NS - 1)
            def _():
                pltpu.sync_copy(rows_v.at[pl.ds(0, tail)],
                                acc_sp.at[pl.ds(nfull * WIN, tail)])
        plsc.subcore_barrier()

        def do_window(j, buf, sem, first):
            @pl.when(jnp.logical_not(first))
            def _():
                pltpu.make_async_copy(
                    buf, acc_sp.at[dst_v.at[j - 2]], sem).wait()
            pltpu.sync_copy(h_h.at[c].at[src_v.at[j]], buf)

            def rowscale(g, carry2):
                p16 = p_v[j, pl.ds(g * LANES, LANES)]
                for r in range(LANES):
                    a = p16[r]
                    row = g * LANES + r
                    for q in range(cl):
                        sl = pl.ds(q * LANES, LANES)
                        buf[row, sl] = buf[row, sl] * a
                return carry2
            lax.fori_loop(0, WIN // LANES, rowscale, 0)
            pltpu.async_copy(buf, acc_sp.at[dst_v.at[j]], sem)

        def winpair(i, carry):
            do_window(2 * i, rows_v, ssem0, i == 0)
            do_window(2 * i + 1, rows2_v, ssem1, i == 0)
            return carry
        lax.fori_loop(0, wpt // 2, winpair, 0)
        pltpu.make_async_copy(rows_v, acc_sp.at[dst_v.at[wpt - 2]],
                              ssem0).wait()
        pltpu.make_async_copy(rows2_v, acc_sp.at[dst_v.at[wpt - 1]],
                              ssem1).wait()
        plsc.subcore_barrier()

        def read_chunk(base, nrows):
            sl = pl.ds(base, nrows)
            pltpu.sync_copy(acc_sp.at[sl], rows_v.at[pl.ds(0, nrows)])
            pltpu.sync_copy(dp_h.at[:, sl], dpw_v.at[:, pl.ds(0, nrows)])

            def divgrp(g, carry2):
                gsl = pl.ds(g * LANES, LANES)

                def racc(r, acc):
                    return acc + dpw_v[r, gsl]
                d16 = lax.fori_loop(0, NC * NS, racc,
                                    jnp.zeros((LANES,), jnp.float32))
                inv = 1.0 / (d16 + 1e-16)
                for r in range(LANES):
                    iv = inv[r]
                    row = g * LANES + r
                    for q in range(cl):
                        qsl = pl.ds(q * LANES, LANES)
                        rows_v[row, qsl] = rows_v[row, qsl] * iv
                return carry2
            lax.fori_loop(0, nrows // LANES, divgrp, 0)
            pltpu.sync_copy(rows_v.at[pl.ds(0, nrows)], out_h.at[c, sl])

        for i in range(rounds):
            ch = s + NS * i

            @pl.when(ch < nfull)
            def _():
                read_chunk(ch * WIN, WIN)
        if tail:
            @pl.when(s == NS - 1)
            def _():
                read_chunk(nfull * WIN, tail)

    return k(hsplit, dparts, p2, src2, dst2)


def kernel(x, edge_index, W1, att_src1, att_dst1, b1,
           W2, att_src2, att_dst2, b2):
    n, _ = x.shape
    e = edge_index.shape[1]
    npad = _round_up(n + 1, 2 * LANES)
    epad = _round_up(e + n, NC * NS * WIN)
    nw = NC * NS
    wpt = epad // (nw * WIN)

    xp = jnp.zeros((npad, x.shape[1]), jnp.float32).at[:n].set(x)
    loop = jnp.arange(n, dtype=jnp.int32)
    pad = jnp.full((epad - e - n,), n, jnp.int32)
    src = jnp.concatenate(
        [edge_index[0].astype(jnp.int32), loop, pad]).reshape(nw, wpt, WIN)
    dst = jnp.concatenate(
        [edge_index[1].astype(jnp.int32), loop, pad]).reshape(nw, wpt, WIN)

    srcb = src.reshape(NS, nw * wpt // NS, WIN)
    dstb = dst.reshape(NS, nw * wpt // NS, WIN)

    h1, as1, ad1 = _tc_embed(xp, W1, att_src1[0], att_dst1[0])
    p1, dp1 = _sc_denom(as1, ad1, src, dst)
    acc1 = _sc_aggregate(h1, dp1, p1.reshape(NS, -1, WIN), srcb, dstb)

    h2, as2, ad2 = _tc_mid(acc1, b1, W2, att_src2[0], att_dst2[0])
    p2, dp2 = _sc_denom(as2, ad2, src, dst)
    acc2 = _sc_aggregate(h2, dp2, p2.reshape(NS, -1, WIN), srcb, dstb)

    out = _tc_out(acc2, b2)
    return out[:n]


# speedup vs baseline: 22.6712x; 1.0012x over previous
import functools

import jax
import jax.numpy as jnp
from jax import lax
from jax.experimental import pallas as pl
from jax.experimental.pallas import tpu as pltpu
from jax.experimental.pallas import tpu_sc as plsc

NC = 2
NS = 16
LANES = 16
WIN = 128


def _round_up(v, m):
    return (v + m - 1) // m * m


_GDN = lax.GatherDimensionNumbers(
    offset_dims=(), collapsed_slice_dims=(0,), start_index_map=(0,))


def _lane_perm(v, perm):
    return lax.gather(v, perm[:, None], _GDN, (1,),
                      mode=lax.GatherScatterMode.PROMISE_IN_BOUNDS)


def _lane_max(v):
    idx = lax.iota(jnp.int32, LANES)
    for sh in (8, 4, 2, 1):
        v = jnp.maximum(v, _lane_perm(v, jnp.bitwise_and(idx + sh, LANES - 1)))
    return v


def _segment_add_16(den_ref, keys, vals):
    lid = lax.iota(jnp.int32, LANES)
    sk, sv = plsc.sort_key_val(keys, vals)
    cum = plsc.cumsum(sv)
    pk = _lane_perm(sk, jnp.maximum(lid - 1, 0))
    isfirst = jnp.logical_or(lid == 0, sk != pk)
    nk = _lane_perm(sk, jnp.minimum(lid + 1, LANES - 1))
    islast = jnp.logical_or(lid == LANES - 1, sk != nk)
    runfirst = plsc.cummax(jnp.where(isfirst, lid, 0))
    prevcum = _lane_perm(cum, jnp.maximum(runfirst - 1, 0))
    prevcum = jnp.where(runfirst == 0, jnp.zeros_like(prevcum), prevcum)
    plsc.addupdate_scatter(den_ref, [sk], cum - prevcum, mask=islast)


def _tc_embed(xp, w, att_s, att_d):
    npad = xp.shape[0]
    cdim = w.shape[1]
    hc = cdim // NC

    def body(x_ref, w_ref, s_ref, d_ref, h_ref, as_ref, ad_ref):
        h = jnp.dot(x_ref[...], w_ref[...], preferred_element_type=jnp.float32)
        h_ref[0] = h[:, :hc]
        h_ref[1] = h[:, hc:]
        as_ref[...] = jnp.sum(h * s_ref[...][None, :], axis=1)
        ad_ref[...] = jnp.sum(h * d_ref[...][None, :], axis=1)

    return pl.pallas_call(
        body,
        out_shape=[
            jax.ShapeDtypeStruct((NC, npad, hc), jnp.float32),
            jax.ShapeDtypeStruct((npad,), jnp.float32),
            jax.ShapeDtypeStruct((npad,), jnp.float32),
        ],
    )(xp, w, att_s, att_d)


def _tc_mid(parts, b, w, att_s, att_d):
    npad = parts.shape[1]
    cdim = w.shape[1]
    hc = cdim // NC

    def body(p_ref, b_ref, w_ref, s_ref, d_ref, h_ref, as_ref, ad_ref):
        g = jnp.concatenate([p_ref[0], p_ref[1]], axis=1)
        x = jax.nn.relu(g + b_ref[...][None, :])
        h = jnp.dot(x, w_ref[...], preferred_element_type=jnp.float32)
        h_ref[0] = h[:, :hc]
        h_ref[1] = h[:, hc:]
        as_ref[...] = jnp.sum(h * s_ref[...][None, :], axis=1)
        ad_ref[...] = jnp.sum(h * d_ref[...][None, :], axis=1)

    return pl.pallas_call(
        body,
        out_shape=[
            jax.ShapeDtypeStruct((NC, npad, hc), jnp.float32),
            jax.ShapeDtypeStruct((npad,), jnp.float32),
            jax.ShapeDtypeStruct((npad,), jnp.float32),
        ],
    )(parts, b, w, att_s, att_d)


def _tc_out(parts, b):
    _, npad, hcdim = parts.shape
    cdim = NC * hcdim

    def body(p_ref, b_ref, o_ref):
        o = jnp.concatenate([p_ref[0], p_ref[1]], axis=1) + b_ref[...][None, :]
        m = jnp.max(o, axis=1, keepdims=True)
        z = o - m
        o_ref[...] = z - jnp.log(jnp.sum(jnp.exp(z), axis=1, keepdims=True))

    return pl.pallas_call(
        body,
        out_shape=jax.ShapeDtypeStruct((npad, cdim), jnp.float32),
    )(parts, b)


def _sc_denom(asv, adv, src2, dst2):
    npad = asv.shape[0]
    nw, wpt, _ = src2.shape
    npt = npad // NS

    mesh = plsc.VectorSubcoreMesh(core_axis_name="c", subcore_axis_name="s")

    @functools.partial(
        pl.kernel,
        out_type=[
            jax.ShapeDtypeStruct((nw, wpt, WIN), jnp.float32),
            jax.ShapeDtypeStruct((nw, npad), jnp.float32),
        ],
        mesh=mesh,
        compiler_params=pltpu.CompilerParams(needs_layout_passes=False, use_tc_tiling_on_sc=False),
        scratch_types=[
            pltpu.VMEM((npad,), jnp.float32),
            pltpu.VMEM((npad,), jnp.float32),
            pltpu.VMEM((wpt, WIN), jnp.int32),
            pltpu.VMEM((wpt, WIN), jnp.int32),
            pltpu.VMEM((wpt, WIN), jnp.float32),
            pltpu.VMEM((npad,), jnp.float32),
        ],
    )
    def k(asv_h, adv_h, src_h, dst_h, p_h, dp_h,
          asv_v, adv_v, src_v, dst_v, p_v, den_v):
        c = lax.axis_index("c")
        s = lax.axis_index("s")
        wid = s * NC + c
        pltpu.sync_copy(asv_h, asv_v)
        pltpu.sync_copy(adv_h, adv_v)
        pltpu.sync_copy(src_h.at[wid], src_v)
        pltpu.sync_copy(dst_h.at[wid], dst_v)

        def zero(i, carry):
            den_v[pl.ds(i * LANES, LANES)] = jnp.zeros((LANES,), jnp.float32)
            return carry
        lax.fori_loop(0, npad // LANES, zero, 0)

        def mx(i, acc):
            return jnp.maximum(acc, asv_v[pl.ds(i * LANES, LANES)])
        acc = lax.fori_loop(0, npad // LANES, mx,
                            jnp.full((LANES,), -jnp.inf, jnp.float32))
        a_top = _lane_max(acc)

        def win(j, carry):
            def grp(g, carry2):
                sl = pl.ds(g * LANES, LANES)
                si = src_v[j, sl]
                di = dst_v[j, sl]
                a_s = plsc.load_gather(asv_v, [si])
                a_d = plsc.load_gather(adv_v, [di])
                t = a_s + a_d
                e = jnp.maximum(t, 0.2 * t)
                u = a_top + a_d
                mp = jnp.maximum(u, 0.2 * u)
                p16 = jnp.exp(e - mp)
                p_v[j, sl] = p16
                _segment_add_16(den_v, di, p16)
                return carry2
            lax.fori_loop(0, WIN // LANES, grp, 0)
            return carry
        lax.fori_loop(0, wpt, win, 0)

        pltpu.sync_copy(p_v, p_h.at[wid])
        pltpu.sync_copy(den_v, dp_h.at[wid])

    return k(asv, adv, src2, dst2)


def _sc_aggregate(hsplit, dparts, p2, src2, dst2):
    _, npad, hc = hsplit.shape
    ns, wpt, _ = src2.shape
    cl = hc // LANES
    nfull = npad // WIN
    tail = npad - nfull * WIN
    rounds = _round_up(nfull + (1 if tail else 0), NS) // NS

    mesh = plsc.VectorSubcoreMesh(core_axis_name="c", subcore_axis_name="s")

    @functools.partial(
        pl.kernel,
        out_type=jax.ShapeDtypeStruct((NC, npad, hc), jnp.float32),
        mesh=mesh,
        compiler_params=pltpu.CompilerParams(needs_layout_passes=False, use_tc_tiling_on_sc=False),
        scratch_types=[
            pltpu.VMEM((NC * NS, WIN), jnp.float32),
            pltpu.VMEM((wpt, WIN), jnp.int32),
            pltpu.VMEM((wpt, WIN), jnp.int32),
            pltpu.VMEM((wpt, WIN), jnp.float32),
            pltpu.VMEM((WIN, hc), jnp.float32),
            pltpu.VMEM((WIN, hc), jnp.float32),
            pltpu.SemaphoreType.DMA,
            pltpu.SemaphoreType.DMA,
            pltpu.VMEM_SHARED((npad, hc), jnp.float32),
        ],
    )
    def k(h_h, dp_h, p_h, src_h, dst_h, out_h,
          dpw_v, src_v, dst_v, p_v, rows_v, rows2_v, ssem0, ssem1, acc_sp):
        c = lax.axis_index("c")
        s = lax.axis_index("s")

        pltpu.sync_copy(src_h.at[s], src_v)
        pltpu.sync_copy(dst_h.at[s], dst_v)
        pltpu.sync_copy(p_h.at[s], p_v)

        def zrow(r, carry):
            for q in range(cl):
                rows_v[r, pl.ds(q * LANES, LANES)] = jnp.zeros(
                    (LANES,), jnp.float32)
            return carry
        lax.fori_loop(0, WIN, zrow, 0)

        for i in range(rounds):
            ch = s + NS * i

            @pl.when(ch < nfull)
            def _():
                pltpu.sync_copy(rows_v, acc_sp.at[pl.ds(ch * WIN, WIN)])
        if tail:
            @pl.when(s == NS - 1)
            def _():
                pltpu.sync_copy(rows_v.at[pl.ds(0, tail)],
                                acc_sp.at[pl.ds(nfull * WIN, tail)])
        plsc.subcore_barrier()

        def do_window(j, buf, sem, first):
            @pl.when(jnp.logical_not(first))
            def _():
                pltpu.make_async_copy(
                    buf, acc_sp.at[dst_v.at[j - 2]], sem).wait()
            pltpu.sync_copy(h_h.at[c].at[src_v.at[j]], buf)

            def rowscale(g, carry2):
                p16 = p_v[j, pl.ds(g * LANES, LANES)]
                for r in range(LANES):
                    a = p16[r]
                    row = g * LANES + r
                    for q in range(cl):
                        sl = pl.ds(q * LANES, LANES)
                        buf[row, sl] = buf[row, sl] * a
                return carry2
            lax.fori_loop(0, WIN // LANES, rowscale, 0)
            pltpu.async_copy(buf, acc_sp.at[dst_v.at[j]], sem, add=True)

        def winpair(i, carry):
            do_window(2 * i, rows_v, ssem0, i == 0)
            do_window(2 * i + 1, rows2_v, ssem1, i == 0)
            return carry
        lax.fori_loop(0, wpt // 2, winpair, 0)
        pltpu.make_async_copy(rows_v, acc_sp.at[dst_v.at[wpt - 2]],
                              ssem0).wait()
        pltpu.make_async_copy(rows2_v, acc_sp.at[dst_v.at[wpt - 1]],
                              ssem1).wait()
        plsc.subcore_barrier()

        def read_chunk(base, nrows):
            sl = pl.ds(base, nrows)
            pltpu.sync_copy(acc_sp.at[sl], rows_v.at[pl.ds(0, nrows)])
            pltpu.sync_copy(dp_h.at[:, sl], dpw_v.at[:, pl.ds(0, nrows)])

            def divgrp(g, carry2):
                gsl = pl.ds(g * LANES, LANES)

                def racc(r, acc):
                    return acc + dpw_v[r, gsl]
                d16 = lax.fori_loop(0, NC * NS, racc,
                                    jnp.zeros((LANES,), jnp.float32))
                inv = 1.0 / (d16 + 1e-16)
                for r in range(LANES):
                    iv = inv[r]
                    row = g * LANES + r
                    for q in range(cl):
                        qsl = pl.ds(q * LANES, LANES)
                        rows_v[row, qsl] = rows_v[row, qsl] * iv
                return carry2
            lax.fori_loop(0, nrows // LANES, divgrp, 0)
            pltpu.sync_copy(rows_v.at[pl.ds(0, nrows)], out_h.at[c, sl])

        for i in range(rounds):
            ch = s + NS * i

            @pl.when(ch < nfull)
            def _():
                read_chunk(ch * WIN, WIN)
        if tail:
            @pl.when(s == NS - 1)
            def _():
                read_chunk(nfull * WIN, tail)

    return k(hsplit, dparts, p2, src2, dst2)


def kernel(x, edge_index, W1, att_src1, att_dst1, b1,
           W2, att_src2, att_dst2, b2):
    n, _ = x.shape
    e = edge_index.shape[1]
    npad = _round_up(n + 1, 2 * LANES)
    epad = _round_up(e + n, NC * NS * WIN)
    nw = NC * NS
    wpt = epad // (nw * WIN)

    xp = jnp.zeros((npad, x.shape[1]), jnp.float32).at[:n].set(x)
    loop = jnp.arange(n, dtype=jnp.int32)
    pad = jnp.full((epad - e - n,), n, jnp.int32)
    src = jnp.concatenate(
        [edge_index[0].astype(jnp.int32), loop, pad]).reshape(nw, wpt, WIN)
    dst = jnp.concatenate(
        [edge_index[1].astype(jnp.int32), loop, pad]).reshape(nw, wpt, WIN)

    srcb = src.reshape(NS, nw * wpt // NS, WIN)
    dstb = dst.reshape(NS, nw * wpt // NS, WIN)

    h1, as1, ad1 = _tc_embed(xp, W1, att_src1[0], att_dst1[0])
    p1, dp1 = _sc_denom(as1, ad1, src, dst)
    acc1 = _sc_aggregate(h1, dp1, p1.reshape(NS, -1, WIN), srcb, dstb)

    h2, as2, ad2 = _tc_mid(acc1, b1, W2, att_src2[0], att_dst2[0])
    p2, dp2 = _sc_denom(as2, ad2, src, dst)
    acc2 = _sc_aggregate(h2, dp2, p2.reshape(NS, -1, WIN), srcb, dstb)

    out = _tc_out(acc2, b2)
    return out[:n]


# speedup vs baseline: 26.5327x; 1.1703x over previous
import functools

import jax
import jax.numpy as jnp
from jax import lax
from jax.experimental import pallas as pl
from jax.experimental.pallas import tpu as pltpu
from jax.experimental.pallas import tpu_sc as plsc

NC = 2
NS = 16
LANES = 16
WIN = 128


def _round_up(v, m):
    return (v + m - 1) // m * m


_GDN = lax.GatherDimensionNumbers(
    offset_dims=(), collapsed_slice_dims=(0,), start_index_map=(0,))


def _lane_perm(v, perm):
    return lax.gather(v, perm[:, None], _GDN, (1,),
                      mode=lax.GatherScatterMode.PROMISE_IN_BOUNDS)


def _lane_max(v):
    idx = lax.iota(jnp.int32, LANES)
    for sh in (8, 4, 2, 1):
        v = jnp.maximum(v, _lane_perm(v, jnp.bitwise_and(idx + sh, LANES - 1)))
    return v


def _segment_add_16(den_ref, keys, vals):
    lid = lax.iota(jnp.int32, LANES)
    sk, sv = plsc.sort_key_val(keys, vals)
    cum = plsc.cumsum(sv)
    pk = _lane_perm(sk, jnp.maximum(lid - 1, 0))
    isfirst = jnp.logical_or(lid == 0, sk != pk)
    nk = _lane_perm(sk, jnp.minimum(lid + 1, LANES - 1))
    islast = jnp.logical_or(lid == LANES - 1, sk != nk)
    runfirst = plsc.cummax(jnp.where(isfirst, lid, 0))
    prevcum = _lane_perm(cum, jnp.maximum(runfirst - 1, 0))
    prevcum = jnp.where(runfirst == 0, jnp.zeros_like(prevcum), prevcum)
    plsc.addupdate_scatter(den_ref, [sk], cum - prevcum, mask=islast)


def _tc_embed(xp, w, att_s, att_d):
    npad = xp.shape[0]
    cdim = w.shape[1]
    hc = cdim // NC

    def body(x_ref, w_ref, s_ref, d_ref, h_ref, as_ref, ad_ref):
        h = jnp.dot(x_ref[...], w_ref[...], preferred_element_type=jnp.float32)
        h_ref[0] = h[:, :hc]
        h_ref[1] = h[:, hc:]
        as_ref[...] = jnp.sum(h * s_ref[...][None, :], axis=1)
        ad_ref[...] = jnp.sum(h * d_ref[...][None, :], axis=1)

    return pl.pallas_call(
        body,
        out_shape=[
            jax.ShapeDtypeStruct((NC, npad, hc), jnp.float32),
            jax.ShapeDtypeStruct((npad,), jnp.float32),
            jax.ShapeDtypeStruct((npad,), jnp.float32),
        ],
    )(xp, w, att_s, att_d)


def _tc_mid(parts, b, w, att_s, att_d):
    npad = parts.shape[1]
    cdim = w.shape[1]
    hc = cdim // NC

    def body(p_ref, b_ref, w_ref, s_ref, d_ref, h_ref, as_ref, ad_ref):
        g = jnp.concatenate([p_ref[0], p_ref[1]], axis=1)
        x = jax.nn.relu(g + b_ref[...][None, :])
        h = jnp.dot(x, w_ref[...], preferred_element_type=jnp.float32)
        h_ref[0] = h[:, :hc]
        h_ref[1] = h[:, hc:]
        as_ref[...] = jnp.sum(h * s_ref[...][None, :], axis=1)
        ad_ref[...] = jnp.sum(h * d_ref[...][None, :], axis=1)

    return pl.pallas_call(
        body,
        out_shape=[
            jax.ShapeDtypeStruct((NC, npad, hc), jnp.float32),
            jax.ShapeDtypeStruct((npad,), jnp.float32),
            jax.ShapeDtypeStruct((npad,), jnp.float32),
        ],
    )(parts, b, w, att_s, att_d)


def _tc_out(parts, b):
    _, npad, hcdim = parts.shape
    cdim = NC * hcdim

    def body(p_ref, b_ref, o_ref):
        o = jnp.concatenate([p_ref[0], p_ref[1]], axis=1) + b_ref[...][None, :]
        m = jnp.max(o, axis=1, keepdims=True)
        z = o - m
        o_ref[...] = z - jnp.log(jnp.sum(jnp.exp(z), axis=1, keepdims=True))

    return pl.pallas_call(
        body,
        out_shape=jax.ShapeDtypeStruct((npad, cdim), jnp.float32),
    )(parts, b)


def _sc_denom(asv, adv, src2, dst2):
    npad = asv.shape[0]
    nw, wpt, _ = src2.shape
    npt = npad // NS

    mesh = plsc.VectorSubcoreMesh(core_axis_name="c", subcore_axis_name="s")

    @functools.partial(
        pl.kernel,
        out_type=[
            jax.ShapeDtypeStruct((nw, wpt, WIN), jnp.float32),
            jax.ShapeDtypeStruct((nw, npad), jnp.float32),
        ],
        mesh=mesh,
        compiler_params=pltpu.CompilerParams(needs_layout_passes=False, use_tc_tiling_on_sc=False),
        scratch_types=[
            pltpu.VMEM((npad,), jnp.float32),
            pltpu.VMEM((npad,), jnp.float32),
            pltpu.VMEM((wpt, WIN), jnp.int32),
            pltpu.VMEM((wpt, WIN), jnp.int32),
            pltpu.VMEM((wpt, WIN), jnp.float32),
            pltpu.VMEM((npad,), jnp.float32),
        ],
    )
    def k(asv_h, adv_h, src_h, dst_h, p_h, dp_h,
          asv_v, adv_v, src_v, dst_v, p_v, den_v):
        c = lax.axis_index("c")
        s = lax.axis_index("s")
        wid = s * NC + c
        pltpu.sync_copy(asv_h, asv_v)
        pltpu.sync_copy(adv_h, adv_v)
        pltpu.sync_copy(src_h.at[wid], src_v)
        pltpu.sync_copy(dst_h.at[wid], dst_v)

        def zero(i, carry):
            den_v[pl.ds(i * LANES, LANES)] = jnp.zeros((LANES,), jnp.float32)
            return carry
        lax.fori_loop(0, npad // LANES, zero, 0)

        def mx(i, acc):
            return jnp.maximum(acc, asv_v[pl.ds(i * LANES, LANES)])
        acc = lax.fori_loop(0, npad // LANES, mx,
                            jnp.full((LANES,), -jnp.inf, jnp.float32))
        a_top = _lane_max(acc)

        def win(j, carry):
            def grp(g, carry2):
                sl = pl.ds(g * LANES, LANES)
                si = src_v[j, sl]
                di = dst_v[j, sl]
                a_s = plsc.load_gather(asv_v, [si])
                a_d = plsc.load_gather(adv_v, [di])
                t = a_s + a_d
                e = jnp.maximum(t, 0.2 * t)
                u = a_top + a_d
                mp = jnp.maximum(u, 0.2 * u)
                p16 = jnp.exp(e - mp)
                p_v[j, sl] = p16
                _segment_add_16(den_v, di, p16)
                return carry2
            lax.fori_loop(0, WIN // LANES, grp, 0)
            return carry
        lax.fori_loop(0, wpt, win, 0)

        pltpu.sync_copy(p_v, p_h.at[wid])
        pltpu.sync_copy(den_v, dp_h.at[wid])

    return k(asv, adv, src2, dst2)


def _sc_aggregate(hsplit, dparts, p2, src2, dst2):
    _, npad, hc = hsplit.shape
    ns, wpt, _ = src2.shape
    cl = hc // LANES
    nfull = npad // WIN
    tail = npad - nfull * WIN
    rounds = _round_up(nfull + (1 if tail else 0), NS) // NS

    mesh = plsc.VectorSubcoreMesh(core_axis_name="c", subcore_axis_name="s")

    @functools.partial(
        pl.kernel,
        out_type=jax.ShapeDtypeStruct((NC, npad, hc), jnp.float32),
        mesh=mesh,
        compiler_params=pltpu.CompilerParams(needs_layout_passes=False, use_tc_tiling_on_sc=False),
        scratch_types=[
            pltpu.VMEM((NC * NS, WIN), jnp.float32),
            pltpu.VMEM((wpt, WIN), jnp.int32),
            pltpu.VMEM((wpt, WIN), jnp.int32),
            pltpu.VMEM((wpt, WIN), jnp.float32),
            pltpu.VMEM((WIN, hc), jnp.float32),
            pltpu.VMEM((WIN, hc), jnp.float32),
            pltpu.SemaphoreType.DMA,
            pltpu.SemaphoreType.DMA,
            pltpu.SemaphoreType.DMA,
            pltpu.SemaphoreType.DMA,
            pltpu.VMEM_SHARED((npad, hc), jnp.float32),
        ],
    )
    def k(h_h, dp_h, p_h, src_h, dst_h, out_h,
          dpw_v, src_v, dst_v, p_v, rows_v, rows2_v,
          gsem0, gsem1, ssem0, ssem1, acc_sp):
        c = lax.axis_index("c")
        s = lax.axis_index("s")

        pltpu.sync_copy(src_h.at[s], src_v)
        pltpu.sync_copy(dst_h.at[s], dst_v)
        pltpu.sync_copy(p_h.at[s], p_v)

        def zrow(r, carry):
            for q in range(cl):
                rows_v[r, pl.ds(q * LANES, LANES)] = jnp.zeros(
                    (LANES,), jnp.float32)
            return carry
        lax.fori_loop(0, WIN, zrow, 0)

        for i in range(rounds):
            ch = s + NS * i

            @pl.when(ch < nfull)
            def _():
                pltpu.sync_copy(rows_v, acc_sp.at[pl.ds(ch * WIN, WIN)])
        if tail:
            @pl.when(s == NS - 1)
            def _():
                pltpu.sync_copy(rows_v.at[pl.ds(0, tail)],
                                acc_sp.at[pl.ds(nfull * WIN, tail)])
        plsc.subcore_barrier()

        def stage(j, buf, gsem, sbsem, obuf, gosem, sosem, first, last):
            pltpu.make_async_copy(h_h.at[c].at[src_v.at[j]],
                                  buf, gsem).wait()

            @pl.when(jnp.logical_not(first))
            def _():
                pltpu.make_async_copy(
                    obuf, acc_sp.at[dst_v.at[j - 1]], sosem).wait()

            @pl.when(jnp.logical_not(last))
            def _():
                pltpu.async_copy(h_h.at[c].at[src_v.at[j + 1]], obuf, gosem)

            def rowscale(g, carry2):
                p16 = p_v[j, pl.ds(g * LANES, LANES)]
                for r in range(LANES):
                    a = p16[r]
                    row = g * LANES + r
                    for q in range(cl):
                        sl = pl.ds(q * LANES, LANES)
                        buf[row, sl] = buf[row, sl] * a
                return carry2
            lax.fori_loop(0, WIN // LANES, rowscale, 0)
            pltpu.async_copy(buf, acc_sp.at[dst_v.at[j]], sbsem, add=True)

        def winpair(i, carry):
            stage(2 * i, rows_v, gsem0, ssem0, rows2_v, gsem1, ssem1,
                  i == 0, jnp.bool_(False))
            stage(2 * i + 1, rows2_v, gsem1, ssem1, rows_v, gsem0, ssem0,
                  jnp.bool_(False), i == wpt // 2 - 1)
            return carry
        pltpu.async_copy(h_h.at[c].at[src_v.at[0]], rows_v, gsem0)
        lax.fori_loop(0, wpt // 2, winpair, 0)
        pltpu.make_async_copy(rows2_v, acc_sp.at[dst_v.at[wpt - 1]],
                              ssem1).wait()
        plsc.subcore_barrier()

        def read_chunk(base, nrows):
            sl = pl.ds(base, nrows)
            pltpu.sync_copy(acc_sp.at[sl], rows_v.at[pl.ds(0, nrows)])
            pltpu.sync_copy(dp_h.at[:, sl], dpw_v.at[:, pl.ds(0, nrows)])

            def divgrp(g, carry2):
                gsl = pl.ds(g * LANES, LANES)

                def racc(r, acc):
                    return acc + dpw_v[r, gsl]
                d16 = lax.fori_loop(0, NC * NS, racc,
                                    jnp.zeros((LANES,), jnp.float32))
                inv = 1.0 / (d16 + 1e-16)
                for r in range(LANES):
                    iv = inv[r]
                    row = g * LANES + r
                    for q in range(cl):
                        qsl = pl.ds(q * LANES, LANES)
                        rows_v[row, qsl] = rows_v[row, qsl] * iv
                return carry2
            lax.fori_loop(0, nrows // LANES, divgrp, 0)
            pltpu.sync_copy(rows_v.at[pl.ds(0, nrows)], out_h.at[c, sl])

        for i in range(rounds):
            ch = s + NS * i

            @pl.when(ch < nfull)
            def _():
                read_chunk(ch * WIN, WIN)
        if tail:
            @pl.when(s == NS - 1)
            def _():
                read_chunk(nfull * WIN, tail)

    return k(hsplit, dparts, p2, src2, dst2)


def kernel(x, edge_index, W1, att_src1, att_dst1, b1,
           W2, att_src2, att_dst2, b2):
    n, _ = x.shape
    e = edge_index.shape[1]
    npad = _round_up(n + 1, 2 * LANES)
    epad = _round_up(e + n, NC * NS * WIN)
    nw = NC * NS
    wpt = epad // (nw * WIN)

    xp = jnp.zeros((npad, x.shape[1]), jnp.float32).at[:n].set(x)
    loop = jnp.arange(n, dtype=jnp.int32)
    pad = jnp.full((epad - e - n,), n, jnp.int32)
    src = jnp.concatenate(
        [edge_index[0].astype(jnp.int32), loop, pad]).reshape(nw, wpt, WIN)
    dst = jnp.concatenate(
        [edge_index[1].astype(jnp.int32), loop, pad]).reshape(nw, wpt, WIN)

    srcb = src.reshape(NS, nw * wpt // NS, WIN)
    dstb = dst.reshape(NS, nw * wpt // NS, WIN)

    h1, as1, ad1 = _tc_embed(xp, W1, att_src1[0], att_dst1[0])
    p1, dp1 = _sc_denom(as1, ad1, src, dst)
    acc1 = _sc_aggregate(h1, dp1, p1.reshape(NS, -1, WIN), srcb, dstb)

    h2, as2, ad2 = _tc_mid(acc1, b1, W2, att_src2[0], att_dst2[0])
    p2, dp2 = _sc_denom(as2, ad2, src, dst)
    acc2 = _sc_aggregate(h2, dp2, p2.reshape(NS, -1, WIN), srcb, dstb)

    out = _tc_out(acc2, b2)
    return out[:n]


# speedup vs baseline: 30.2684x; 1.1408x over previous
import functools

import jax
import jax.numpy as jnp
from jax import lax
from jax.experimental import pallas as pl
from jax.experimental.pallas import tpu as pltpu
from jax.experimental.pallas import tpu_sc as plsc

NC = 2
NS = 16
LANES = 16
WIN = 128


def _round_up(v, m):
    return (v + m - 1) // m * m


_GDN = lax.GatherDimensionNumbers(
    offset_dims=(), collapsed_slice_dims=(0,), start_index_map=(0,))


def _lane_perm(v, perm):
    return lax.gather(v, perm[:, None], _GDN, (1,),
                      mode=lax.GatherScatterMode.PROMISE_IN_BOUNDS)


def _lane_max(v):
    idx = lax.iota(jnp.int32, LANES)
    for sh in (8, 4, 2, 1):
        v = jnp.maximum(v, _lane_perm(v, jnp.bitwise_and(idx + sh, LANES - 1)))
    return v


def _segment_add_16(den_ref, keys, vals):
    lid = lax.iota(jnp.int32, LANES)
    sk, sv = plsc.sort_key_val(keys, vals)
    cum = plsc.cumsum(sv)
    pk = _lane_perm(sk, jnp.maximum(lid - 1, 0))
    isfirst = jnp.logical_or(lid == 0, sk != pk)
    nk = _lane_perm(sk, jnp.minimum(lid + 1, LANES - 1))
    islast = jnp.logical_or(lid == LANES - 1, sk != nk)
    runfirst = plsc.cummax(jnp.where(isfirst, lid, 0))
    prevcum = _lane_perm(cum, jnp.maximum(runfirst - 1, 0))
    prevcum = jnp.where(runfirst == 0, jnp.zeros_like(prevcum), prevcum)
    plsc.addupdate_scatter(den_ref, [sk], cum - prevcum, mask=islast)


def _tc_embed(xp, w, att_s, att_d):
    npad = xp.shape[0]
    cdim = w.shape[1]
    hc = cdim // NC

    def body(x_ref, w_ref, s_ref, d_ref, h_ref, as_ref, ad_ref):
        h = jnp.dot(x_ref[...], w_ref[...], preferred_element_type=jnp.float32)
        h_ref[0] = h[:, :hc]
        h_ref[1] = h[:, hc:]
        as_ref[...] = jnp.sum(h * s_ref[...][None, :], axis=1)
        ad_ref[...] = jnp.sum(h * d_ref[...][None, :], axis=1)

    return pl.pallas_call(
        body,
        out_shape=[
            jax.ShapeDtypeStruct((NC, npad, hc), jnp.float32),
            jax.ShapeDtypeStruct((npad,), jnp.float32),
            jax.ShapeDtypeStruct((npad,), jnp.float32),
        ],
    )(xp, w, att_s, att_d)


def _tc_mid(parts, b, w, att_s, att_d):
    npad = parts.shape[1]
    cdim = w.shape[1]
    hc = cdim // NC

    def body(p_ref, b_ref, w_ref, s_ref, d_ref, h_ref, as_ref, ad_ref):
        g = jnp.concatenate([p_ref[0], p_ref[1]], axis=1)
        x = jax.nn.relu(g + b_ref[...][None, :])
        h = jnp.dot(x, w_ref[...], preferred_element_type=jnp.float32)
        h_ref[0] = h[:, :hc]
        h_ref[1] = h[:, hc:]
        as_ref[...] = jnp.sum(h * s_ref[...][None, :], axis=1)
        ad_ref[...] = jnp.sum(h * d_ref[...][None, :], axis=1)

    return pl.pallas_call(
        body,
        out_shape=[
            jax.ShapeDtypeStruct((NC, npad, hc), jnp.float32),
            jax.ShapeDtypeStruct((npad,), jnp.float32),
            jax.ShapeDtypeStruct((npad,), jnp.float32),
        ],
    )(parts, b, w, att_s, att_d)


def _tc_out(parts, b):
    _, npad, hcdim = parts.shape
    cdim = NC * hcdim

    def body(p_ref, b_ref, o_ref):
        o = jnp.concatenate([p_ref[0], p_ref[1]], axis=1) + b_ref[...][None, :]
        m = jnp.max(o, axis=1, keepdims=True)
        z = o - m
        o_ref[...] = z - jnp.log(jnp.sum(jnp.exp(z), axis=1, keepdims=True))

    return pl.pallas_call(
        body,
        out_shape=jax.ShapeDtypeStruct((npad, cdim), jnp.float32),
    )(parts, b)


def _sc_denom(asv, adv, src2, dst2):
    npad = asv.shape[0]
    nw, wpt, _ = src2.shape
    npt = npad // NS

    mesh = plsc.VectorSubcoreMesh(core_axis_name="c", subcore_axis_name="s")

    @functools.partial(
        pl.kernel,
        out_type=[
            jax.ShapeDtypeStruct((nw, wpt, WIN), jnp.float32),
            jax.ShapeDtypeStruct((nw, npad), jnp.float32),
        ],
        mesh=mesh,
        compiler_params=pltpu.CompilerParams(needs_layout_passes=False, use_tc_tiling_on_sc=False),
        scratch_types=[
            pltpu.VMEM((npad,), jnp.float32),
            pltpu.VMEM((npad,), jnp.float32),
            pltpu.VMEM((wpt, WIN), jnp.int32),
            pltpu.VMEM((wpt, WIN), jnp.int32),
            pltpu.VMEM((wpt, WIN), jnp.float32),
            pltpu.VMEM((npad,), jnp.float32),
        ],
    )
    def k(asv_h, adv_h, src_h, dst_h, p_h, dp_h,
          asv_v, adv_v, src_v, dst_v, p_v, den_v):
        c = lax.axis_index("c")
        s = lax.axis_index("s")
        wid = s * NC + c
        pltpu.sync_copy(asv_h, asv_v)
        pltpu.sync_copy(adv_h, adv_v)
        pltpu.sync_copy(src_h.at[wid], src_v)
        pltpu.sync_copy(dst_h.at[wid], dst_v)

        def zero(i, carry):
            den_v[pl.ds(i * LANES, LANES)] = jnp.zeros((LANES,), jnp.float32)
            return carry
        lax.fori_loop(0, npad // LANES, zero, 0)

        def mx(i, acc):
            return jnp.maximum(acc, asv_v[pl.ds(i * LANES, LANES)])
        acc = lax.fori_loop(0, npad // LANES, mx,
                            jnp.full((LANES,), -jnp.inf, jnp.float32))
        a_top = _lane_max(acc)

        def win(j, carry):
            def grp(g, carry2):
                sl = pl.ds(g * LANES, LANES)
                si = src_v[j, sl]
                di = dst_v[j, sl]
                a_s = plsc.load_gather(asv_v, [si])
                a_d = plsc.load_gather(adv_v, [di])
                t = a_s + a_d
                e = jnp.maximum(t, 0.2 * t)
                u = a_top + a_d
                mp = jnp.maximum(u, 0.2 * u)
                p16 = jnp.exp(e - mp)
                p_v[j, sl] = p16
                _segment_add_16(den_v, di, p16)
                return carry2
            lax.fori_loop(0, WIN // LANES, grp, 0)
            return carry
        lax.fori_loop(0, wpt, win, 0)

        pltpu.sync_copy(p_v, p_h.at[wid])
        pltpu.sync_copy(den_v, dp_h.at[wid])

    return k(asv, adv, src2, dst2)


def _sc_aggregate(hsplit, dparts, p2, src2, dst2):
    _, npad, hc = hsplit.shape
    ns, wpt, _ = src2.shape
    cl = hc // LANES
    nfull = npad // WIN
    tail = npad - nfull * WIN
    rounds = _round_up(nfull + (1 if tail else 0), NS) // NS

    mesh = plsc.VectorSubcoreMesh(core_axis_name="c", subcore_axis_name="s")

    @functools.partial(
        pl.kernel,
        out_type=jax.ShapeDtypeStruct((NC, npad, hc), jnp.float32),
        mesh=mesh,
        compiler_params=pltpu.CompilerParams(needs_layout_passes=False, use_tc_tiling_on_sc=False),
        scratch_types=[
            pltpu.VMEM((NC * NS, WIN), jnp.float32),
            pltpu.VMEM((wpt, WIN), jnp.int32),
            pltpu.VMEM((wpt, WIN), jnp.int32),
            pltpu.VMEM((wpt, WIN), jnp.float32),
            pltpu.VMEM((WIN, hc), jnp.float32),
            pltpu.VMEM((WIN, hc), jnp.float32),
            pltpu.VMEM((WIN, hc), jnp.float32),
            pltpu.SemaphoreType.DMA,
            pltpu.SemaphoreType.DMA,
            pltpu.SemaphoreType.DMA,
            pltpu.SemaphoreType.DMA,
            pltpu.SemaphoreType.DMA,
            pltpu.SemaphoreType.DMA,
            pltpu.VMEM_SHARED((npad, hc), jnp.float32),
        ],
    )
    def k(h_h, dp_h, p_h, src_h, dst_h, out_h,
          dpw_v, src_v, dst_v, p_v, rows_v, rows2_v, rows3_v,
          gsem0, gsem1, gsem2, ssem0, ssem1, ssem2, acc_sp):
        c = lax.axis_index("c")
        s = lax.axis_index("s")

        pltpu.sync_copy(src_h.at[s], src_v)
        pltpu.sync_copy(dst_h.at[s], dst_v)
        pltpu.sync_copy(p_h.at[s], p_v)

        def zrow(r, carry):
            for q in range(cl):
                rows_v[r, pl.ds(q * LANES, LANES)] = jnp.zeros(
                    (LANES,), jnp.float32)
            return carry
        lax.fori_loop(0, WIN, zrow, 0)

        for i in range(rounds):
            ch = s + NS * i

            @pl.when(ch < nfull)
            def _():
                pltpu.sync_copy(rows_v, acc_sp.at[pl.ds(ch * WIN, WIN)])
        if tail:
            @pl.when(s == NS - 1)
            def _():
                pltpu.sync_copy(rows_v.at[pl.ds(0, tail)],
                                acc_sp.at[pl.ds(nfull * WIN, tail)])
        plsc.subcore_barrier()

        bufs = (rows_v, rows2_v, rows3_v)
        gsems = (gsem0, gsem1, gsem2)
        ssems = (ssem0, ssem1, ssem2)
        nbuf = 3

        def stage(j, k, first):
            buf = bufs[k]
            kn = (k + 2) % nbuf
            pltpu.make_async_copy(h_h.at[c].at[src_v.at[j]],
                                  buf, gsems[k]).wait()

            def rowscale(g, carry2):
                p16 = p_v[j, pl.ds(g * LANES, LANES)]
                for r in range(LANES):
                    a = p16[r]
                    row = g * LANES + r
                    for q in range(cl):
                        sl = pl.ds(q * LANES, LANES)
                        buf[row, sl] = buf[row, sl] * a
                return carry2
            lax.fori_loop(0, WIN // LANES, rowscale, 0)

            @pl.when(jnp.logical_not(first))
            def _():
                pltpu.make_async_copy(
                    bufs[kn], acc_sp.at[dst_v.at[j - 1]], ssems[kn]).wait()

            @pl.when(j + 2 < wpt)
            def _():
                pltpu.async_copy(h_h.at[c].at[src_v.at[j + 2]],
                                 bufs[kn], gsems[kn])
            pltpu.async_copy(buf, acc_sp.at[dst_v.at[j]], ssems[k], add=True)

        def wintrip(i, carry):
            stage(nbuf * i, 0, i == 0)
            stage(nbuf * i + 1, 1, jnp.bool_(False))
            stage(nbuf * i + 2, 2, jnp.bool_(False))
            return carry
        pltpu.async_copy(h_h.at[c].at[src_v.at[0]], rows_v, gsem0)
        pltpu.async_copy(h_h.at[c].at[src_v.at[1]], rows2_v, gsem1)
        lax.fori_loop(0, wpt // nbuf, wintrip, 0)
        pltpu.make_async_copy(bufs[(wpt - 1) % nbuf],
                              acc_sp.at[dst_v.at[wpt - 1]],
                              ssems[(wpt - 1) % nbuf]).wait()
        plsc.subcore_barrier()

        def read_chunk(base, nrows):
            sl = pl.ds(base, nrows)
            pltpu.sync_copy(acc_sp.at[sl], rows_v.at[pl.ds(0, nrows)])
            pltpu.sync_copy(dp_h.at[:, sl], dpw_v.at[:, pl.ds(0, nrows)])

            def divgrp(g, carry2):
                gsl = pl.ds(g * LANES, LANES)

                def racc(r, acc):
                    return acc + dpw_v[r, gsl]
                d16 = lax.fori_loop(0, NC * NS, racc,
                                    jnp.zeros((LANES,), jnp.float32))
                inv = 1.0 / (d16 + 1e-16)
                for r in range(LANES):
                    iv = inv[r]
                    row = g * LANES + r
                    for q in range(cl):
                        qsl = pl.ds(q * LANES, LANES)
                        rows_v[row, qsl] = rows_v[row, qsl] * iv
                return carry2
            lax.fori_loop(0, nrows // LANES, divgrp, 0)
            pltpu.sync_copy(rows_v.at[pl.ds(0, nrows)], out_h.at[c, sl])

        for i in range(rounds):
            ch = s + NS * i

            @pl.when(ch < nfull)
            def _():
                read_chunk(ch * WIN, WIN)
        if tail:
            @pl.when(s == NS - 1)
            def _():
                read_chunk(nfull * WIN, tail)

    return k(hsplit, dparts, p2, src2, dst2)


def kernel(x, edge_index, W1, att_src1, att_dst1, b1,
           W2, att_src2, att_dst2, b2):
    n, _ = x.shape
    e = edge_index.shape[1]
    npad = _round_up(n + 1, 2 * LANES)
    epad = _round_up(e + n, 3 * NC * NS * WIN)
    nw = NC * NS
    wpt = epad // (nw * WIN)

    xp = jnp.zeros((npad, x.shape[1]), jnp.float32).at[:n].set(x)
    loop = jnp.arange(n, dtype=jnp.int32)
    pad = jnp.full((epad - e - n,), n, jnp.int32)
    src = jnp.concatenate(
        [edge_index[0].astype(jnp.int32), loop, pad]).reshape(nw, wpt, WIN)
    dst = jnp.concatenate(
        [edge_index[1].astype(jnp.int32), loop, pad]).reshape(nw, wpt, WIN)

    srcb = src.reshape(NS, nw * wpt // NS, WIN)
    dstb = dst.reshape(NS, nw * wpt // NS, WIN)

    h1, as1, ad1 = _tc_embed(xp, W1, att_src1[0], att_dst1[0])
    p1, dp1 = _sc_denom(as1, ad1, src, dst)
    acc1 = _sc_aggregate(h1, dp1, p1.reshape(NS, -1, WIN), srcb, dstb)

    h2, as2, ad2 = _tc_mid(acc1, b1, W2, att_src2[0], att_dst2[0])
    p2, dp2 = _sc_denom(as2, ad2, src, dst)
    acc2 = _sc_aggregate(h2, dp2, p2.reshape(NS, -1, WIN), srcb, dstb)

    out = _tc_out(acc2, b2)
    return out[:n]
